# baseline (speedup 1.0000x reference)
import functools

import jax
import jax.numpy as jnp
from jax import lax
from jax.experimental import pallas as pl
from jax.experimental.pallas import tpu as pltpu
from jax.experimental.pallas import tpu_sc as plsc

B, S, H = 1024, 200, 768
LANES = 16
NVREG = H // LANES
CHUNK = 16
TGROUP = 8
EPS = 1e-12


def _lane_sum(v):
    dnums = lax.GatherDimensionNumbers(
        offset_dims=(), collapsed_slice_dims=(0,), start_index_map=(0,))
    for shift in (8, 4, 2, 1):
        perm = jnp.arange(LANES, dtype=jnp.int32) ^ shift
        shuffled = lax.gather(
            v, perm[:, None], dimension_numbers=dnums, slice_sizes=(1,),
            mode=lax.GatherScatterMode.PROMISE_IN_BOUNDS)
        v = v + shuffled
    return v


def _rsqrt_vec(v):
    i = lax.bitcast_convert_type(v, jnp.int32)
    i = jnp.int32(0x5F3759DF) - (i >> 1)
    y = lax.bitcast_convert_type(i, jnp.float32)
    half = v * 0.5
    for _ in range(3):
        y = y * (1.5 - half * y * y)
    return y


def _build_kernel(num_cores, num_subcores):
    nw = num_cores * num_subcores
    tokens = B * S
    per_w = tokens // nw
    n_chunks = per_w // CHUNK
    mesh = plsc.VectorSubcoreMesh(core_axis_name="c", subcore_axis_name="s")

    @functools.partial(
        pl.kernel,
        mesh=mesh,
        out_type=jax.ShapeDtypeStruct((tokens, H), jnp.float32),
        scratch_types=(
            [pltpu.VMEM((CHUNK,), jnp.int32) for _ in range(2)]
            + [pltpu.VMEM((CHUNK,), jnp.int32) for _ in range(2)]
            + [pltpu.VMEM((CHUNK, H), jnp.float32) for _ in range(2)]
            + [pltpu.VMEM((CHUNK, H), jnp.float32) for _ in range(2)]
            + [pltpu.VMEM((CHUNK, H), jnp.float32) for _ in range(2)]
            + [pltpu.VMEM((CHUNK, H), jnp.float32) for _ in range(2)]
            + [pltpu.VMEM((H,), jnp.float32) for _ in range(2)]
            + [pltpu.SemaphoreType.DMA for _ in range(12)]
        ),
    )
    def emb_kernel(ids_hbm, tids_hbm, tok_hbm, pos_hbm, typ_hbm, gamma_hbm,
                   beta_hbm, out_hbm,
                   idtok0, idtok1, idtyp0, idtyp1, tokb0, tokb1, typb0, typb1,
                   posb0, posb1, ob0, ob1, g_v, b_v,
                   s_gt0, s_gt1, s_gy0, s_gy1, s_gp0, s_gp1,
                   s_it0, s_it1, s_iy0, s_iy1, s_o0, s_o1):
        idtok = (idtok0, idtok1)
        idtyp = (idtyp0, idtyp1)
        tokb = (tokb0, tokb1)
        typb = (typb0, typb1)
        posb = (posb0, posb1)
        ob = (ob0, ob1)
        s_gt = (s_gt0, s_gt1)
        s_gy = (s_gy0, s_gy1)
        s_gp = (s_gp0, s_gp1)
        s_it = (s_it0, s_it1)
        s_iy = (s_iy0, s_iy1)
        s_o = (s_o0, s_o1)

        wid = lax.axis_index("s") * num_cores + lax.axis_index("c")
        wbase = wid * per_w
        pltpu.sync_copy(gamma_hbm, g_v)
        pltpu.sync_copy(beta_hbm, b_v)

        def pos_idx(k):
            return lax.rem(k * CHUNK + jnp.arange(LANES, dtype=jnp.int32), S)

        def issue_ids(k, p):
            base = wbase + k * CHUNK
            pltpu.async_copy(ids_hbm.at[pl.ds(base, CHUNK)], idtok[p], s_it[p])
            pltpu.async_copy(tids_hbm.at[pl.ds(base, CHUNK)], idtyp[p], s_iy[p])

        def wait_ids(p):
            pltpu.make_async_copy(ids_hbm.at[pl.ds(0, CHUNK)], idtok[p],
                                  s_it[p]).wait()
            pltpu.make_async_copy(tids_hbm.at[pl.ds(0, CHUNK)], idtyp[p],
                                  s_iy[p]).wait()

        def issue_gathers(k, p):
            pltpu.async_copy(tok_hbm.at[idtok[p]], tokb[p], s_gt[p])
            pltpu.async_copy(typ_hbm.at[idtyp[p]], typb[p], s_gy[p])
            pltpu.async_copy(pos_hbm.at[pos_idx(k)], posb[p], s_gp[p])

        def wait_gathers(p):
            pltpu.make_async_copy(tok_hbm.at[idtok[p]], tokb[p], s_gt[p]).wait()
            pltpu.make_async_copy(typ_hbm.at[idtyp[p]], typb[p], s_gy[p]).wait()
            pltpu.make_async_copy(pos_hbm.at[idtok[p]], posb[p], s_gp[p]).wait()

        def wait_out(p):
            pltpu.make_async_copy(ob[p], out_hbm.at[pl.ds(0, CHUNK)],
                                  s_o[p]).wait()

        def compute_group(p, t0):
            tb, yb, pb, o = tokb[p], typb[p], posb[p], ob[p]

            def pass1(j, carry):
                sl = pl.ds(j * LANES, LANES)
                out = []
                out2 = []
                for t in range(TGROUP):
                    c = tb[t0 + t, sl] + yb[t0 + t, sl] + pb[t0 + t, sl]
                    o[t0 + t, sl] = c
                    out.append(carry[t] + c)
                    out2.append(carry[TGROUP + t] + c * c)
                return tuple(out) + tuple(out2)

            zero = jnp.zeros((LANES,), jnp.float32)
            carry = lax.fori_loop(0, NVREG, pass1, (zero,) * (2 * TGROUP),
                                  unroll=2)
            rvs, mrvs = [], []
            for t in range(TGROUP):
                s1 = _lane_sum(carry[t])
                s2 = _lane_sum(carry[TGROUP + t])
                mv = s1 * (1.0 / H)
                var = jnp.maximum(s2 * (1.0 / H) - mv * mv, 0.0)
                rv = _rsqrt_vec(var + EPS)
                rvs.append(rv)
                mrvs.append(mv * rv)

            def pass2(j, carry):
                sl = pl.ds(j * LANES, LANES)
                g = g_v[sl]
                be = b_v[sl]
                for t in range(TGROUP):
                    c = o[t0 + t, sl]
                    o[t0 + t, sl] = (c * rvs[t] - mrvs[t]) * g + be
                return carry

            lax.fori_loop(0, NVREG, pass2, 0, unroll=2)

        def step(k, p):
            wait_gathers(p)
            @pl.when(k + 2 < n_chunks)
            def _():
                issue_ids(k + 2, p)
            q = 1 - p
            @pl.when(k + 1 < n_chunks)
            def _():
                wait_ids(q)
                issue_gathers(k + 1, q)
            @pl.when(k >= 2)
            def _():
                wait_out(p)
            for t0 in range(0, CHUNK, TGROUP):
                compute_group(p, t0)
            pltpu.async_copy(ob[p], out_hbm.at[pl.ds(wbase + k * CHUNK, CHUNK)],
                             s_o[p])

        issue_ids(0, 0)
        issue_ids(1, 1)
        wait_ids(0)
        issue_gathers(0, 0)

        def pair_body(gidx, carry):
            step(2 * gidx, 0)
            step(2 * gidx + 1, 1)
            return carry

        lax.fori_loop(0, n_chunks // 2, pair_body, 0)
        wait_out(0)
        wait_out(1)

    return emb_kernel


def kernel(input_ids, token_type_ids, tok_emb, pos_emb, type_emb, gamma, beta):
    try:
        info = plsc.get_sparse_core_info()
        nc, ns = info.num_cores, info.num_subcores
    except Exception:
        nc, ns = 2, 16
    emb_kernel = _build_kernel(nc, ns)
    flat_ids = input_ids.reshape(-1)
    flat_tids = token_type_ids.reshape(-1)
    out = emb_kernel(flat_ids, flat_tids, tok_emb, pos_emb, type_emb, gamma,
                     beta)
    return out.reshape(B, S, H)
